# baseline (speedup 1.0000x reference)
import functools

import jax
import jax.numpy as jnp
from jax import lax
from jax.experimental import pallas as pl
from jax.experimental.pallas import tpu as pltpu
from jax.experimental.pallas import tpu_sc as plsc

_C = 16
_D = 8
_B = 32
_IN = 64
_H = 16
_W = 16
_OUT = 64
_Q = _OUT * _H * _W
_P4 = 4 * _Q
_EPS = 1e-6
_NW = 32
_CHUNK = _P4 // _NW


def _lsp(x):
    return -jnp.log1p(jnp.exp(-x))


def _sc_hist(x2d, labi):
    mesh = plsc.VectorSubcoreMesh(core_axis_name="c", subcore_axis_name="s")

    @functools.partial(
        pl.kernel,
        out_type=[
            jax.ShapeDtypeStruct((4, _C, _Q), jnp.float32),
            jax.ShapeDtypeStruct((_C,), jnp.float32),
        ],
        mesh=mesh,
        scratch_types=[
            pltpu.VMEM((_B + 16,), jnp.int32),
            pltpu.VMEM((_B, _CHUNK), jnp.float32),
            pltpu.VMEM((_C, _CHUNK), jnp.float32),
            pltpu.VMEM((_C,), jnp.float32),
            pltpu.SemaphoreType.DMA,
        ],
    )
    def hist(x_hbm, lab_hbm, sums_hbm, cnt_hbm, labs_v, xbuf, acc, cntv, sem):
        wid = lax.axis_index("c") * 16 + lax.axis_index("s")
        start = wid * _CHUNK
        ch = start // _Q
        qoff = start % _Q
        cp = pltpu.async_copy(x_hbm.at[:, pl.ds(start, _CHUNK)], xbuf, sem)
        pltpu.sync_copy(lab_hbm, labs_v)
        iot = lax.broadcasted_iota(jnp.int32, (16,), 0)
        zero16 = jnp.zeros((16,), jnp.float32)

        def zbody(i, carry):
            for r in range(_C):
                acc[r, pl.ds(i * 16, 16)] = zero16
            return carry

        lax.fori_loop(0, _CHUNK // 16, zbody, 0)

        lab_lo = labs_v[pl.ds(0, 16)]
        lab_hi = labs_v[pl.ds(16, 16)]
        labs = [lab_lo[i] for i in range(16)] + [lab_hi[i] for i in range(16)]
        cp.wait()

        def bbody(b, carry):
            lab = labs_v[pl.ds(b, 16)][0]
            for j in range(_CHUNK // 16):
                sl = pl.ds(j * 16, 16)
                acc[lab, sl] = acc[lab, sl] + xbuf[b, sl]
            return carry

        lax.fori_loop(0, _B, bbody, 0)
        pltpu.sync_copy(acc, sums_hbm.at[ch, :, pl.ds(qoff, _CHUNK)])

        @pl.when(wid == 0)
        def _():
            cnt = jnp.zeros((16,), jnp.float32)
            for b in range(_B):
                cnt = cnt + jnp.where(iot == labs[b], 1.0, 0.0)
            cntv[...] = cnt
            pltpu.sync_copy(cntv, cnt_hbm)

    return hist(x2d, labi)


def _tc_body(x_ref, s4_ref, cnt_ref, miu_ref, par_ref, out_ref, loss_ref,
             mth_ref, lm_ref, xy0_ref, xy1_ref):
    b = pl.program_id(0)

    @pl.when(b == 0)
    def _stats():
        xw = cnt_ref[...] + _EPS
        w0 = par_ref[33]
        w1 = par_ref[34]
        w2s = par_ref[35]
        mag = (s4_ref[1] + _EPS) / xw
        rot = (s4_ref[0] + _EPS) / xw
        xy0_ref[...] = w2s * ((s4_ref[2] + _EPS) / xw)
        xy1_ref[...] = w2s * ((s4_ref[3] + _EPS) / xw)
        lmag = _lsp(mag + _EPS)
        a_rot = _lsp(rot)
        a_mag = _lsp(mag)
        m_rot = _lsp(miu_ref[:, 0, :])
        m_mag = _lsp(miu_ref[:, 1, :])
        m_mag_e = _lsp(miu_ref[:, 1, :] + _EPS)
        dn = (((1,), (1,)), ((), ()))
        g = (lax.dot_general(a_rot, m_rot, dn,
                             precision=lax.Precision.HIGHEST)
             + lax.dot_general(a_mag, m_mag, dn,
                               precision=lax.Precision.HIGHEST))
        n_a = jnp.sum(a_rot * a_rot + a_mag * a_mag, axis=1,
                      keepdims=True)
        mm = jnp.zeros((_C, _Q), jnp.float32)
        mth0 = jnp.zeros((1, _Q), jnp.float32)
        lossm = jnp.zeros((_D, 128), jnp.float32)
        for d in range(_D):
            al = par_ref[d]
            be = par_ref[8 + d]
            mrd = m_rot[d:d + 1, :]
            mmd = m_mag[d:d + 1, :]
            mth0 = mth0 + be * miu_ref[d, 0:1, :]
            mm = mm + jnp.exp(be * m_mag_e[d:d + 1, :]) * jnp.exp(al * lmag)
            n_m = jnp.sum(mrd * mrd + mmd * mmd)
            nrm = n_a + n_m - 2.0 * g[:, d:d + 1]
            t3 = par_ref[16 + d] / xw
            lossd = par_ref[24 + d] * jnp.mean(nrm + t3)
            rowm = lax.broadcasted_iota(jnp.int32, (_D, 128), 0) == d
            lossm = jnp.where(rowm, lossd, lossm)
        mth_ref[...] = w0 * (par_ref[32] * mag + mth0)
        lm_ref[...] = w1 * jnp.log(mm + _EPS)
        loss_ref[...] = lossm

    x = x_ref[0]
    w0 = par_ref[33]
    w1 = par_ref[34]
    w2s = par_ref[35]
    x0 = w0 * x[:, 0:_Q]
    lx1 = w1 * jnp.log(x[:, _Q:2 * _Q])
    x2 = w2s * x[:, 2 * _Q:3 * _Q]
    x3 = w2s * x[:, 3 * _Q:4 * _Q]
    dr = jnp.abs(x0 - mth_ref[...])
    da = jnp.abs(lx1 - lm_ref[...])
    dxy = (x2 - xy0_ref[...]) ** 2 + (x3 - xy1_ref[...]) ** 2
    dist = dr + da + dxy
    out_ref[0] = jnp.min(dist, axis=0, keepdims=True)


def _tc_main(x2d, sums4, cnt, miu2, params):
    return pl.pallas_call(
        _tc_body,
        grid=(_B,),
        in_specs=[
            pl.BlockSpec((1, 1, _P4), lambda b: (b, 0, 0)),
            pl.BlockSpec((4, _C, _Q), lambda b: (0, 0, 0)),
            pl.BlockSpec((_C, 1), lambda b: (0, 0)),
            pl.BlockSpec((_D, 2, _Q), lambda b: (0, 0, 0)),
            pl.BlockSpec(memory_space=pltpu.SMEM),
        ],
        out_specs=[
            pl.BlockSpec((1, 1, _Q), lambda b: (b, 0, 0)),
            pl.BlockSpec((_D, 128), lambda b: (0, 0)),
        ],
        out_shape=[
            jax.ShapeDtypeStruct((_B, 1, _Q), jnp.float32),
            jax.ShapeDtypeStruct((_D, 128), jnp.float32),
        ],
        scratch_shapes=[pltpu.VMEM((_C, _Q), jnp.float32)] * 4,
    )(x2d, sums4, cnt, miu2, params)


def kernel(x_LE, labels, w1, w2, miu, tao, weight):
    x2d = x_LE.reshape(_B, _P4)
    labi = jnp.pad(labels.astype(jnp.int32), (0, 16))
    sums4, cnt = _sc_hist(x2d, labi)

    w1n = w1 ** 2 / jnp.sum(w1 ** 2)
    t2 = tao ** 2
    ft = t2 / (1.0 + t2)
    fs = 1.0 / (1.0 + t2)
    alpha = w1n * ft
    beta = w1n * fs
    t3coef = (2.0 * _Q) * (t2 ** 2 - 1.0)
    term1 = 1.0 / (1.0 + t2) ** 2
    a_sum = jnp.sum(alpha)[None]
    wsq = weight ** 2
    wpar = jnp.stack([wsq[0], wsq[1], jnp.abs(weight[2])])
    params = jnp.concatenate(
        [alpha, beta, t3coef, term1, a_sum, wpar, jnp.zeros((4,), jnp.float32)]
    ).astype(jnp.float32)

    out2, loss_pad = _tc_main(
        x2d.reshape(_B, 1, _P4), sums4, cnt.reshape(_C, 1),
        miu.reshape(_D, 2, _Q), params)
    out = out2.reshape(_B, _IN, _H, _W)
    loss = loss_pad[:, 0]
    return out, loss


# speedup vs baseline: 1.1931x; 1.1931x over previous
import functools

import jax
import jax.numpy as jnp
from jax import lax
from jax.experimental import pallas as pl
from jax.experimental.pallas import tpu as pltpu
from jax.experimental.pallas import tpu_sc as plsc

_C = 16
_D = 8
_B = 32
_IN = 64
_H = 16
_W = 16
_OUT = 64
_Q = _OUT * _H * _W
_P4 = 4 * _Q
_EPS = 1e-6
_NW = 32
_CHUNK = _P4 // _NW


def _lsp(x):
    return -jnp.log1p(jnp.exp(-x))


def _sc_hist(x2d, labi):
    mesh = plsc.VectorSubcoreMesh(core_axis_name="c", subcore_axis_name="s")

    @functools.partial(
        pl.kernel,
        out_type=[
            jax.ShapeDtypeStruct((4, _C, _Q), jnp.float32),
            jax.ShapeDtypeStruct((_C,), jnp.float32),
        ],
        mesh=mesh,
        scratch_types=[
            pltpu.VMEM((_B + 16,), jnp.int32),
            pltpu.VMEM((_B, _CHUNK), jnp.float32),
            pltpu.VMEM((_C, _CHUNK), jnp.float32),
            pltpu.VMEM((_C,), jnp.float32),
            pltpu.SemaphoreType.DMA,
        ],
    )
    def hist(x_hbm, lab_hbm, sums_hbm, cnt_hbm, labs_v, xbuf, acc, cntv, sem):
        wid = lax.axis_index("c") * 16 + lax.axis_index("s")
        start = wid * _CHUNK
        ch = start // _Q
        qoff = start % _Q
        cp = pltpu.async_copy(x_hbm.at[:, pl.ds(start, _CHUNK)], xbuf, sem)
        pltpu.sync_copy(lab_hbm, labs_v)
        iot = lax.broadcasted_iota(jnp.int32, (16,), 0)
        zero16 = jnp.zeros((16,), jnp.float32)

        def zbody(i, carry):
            for r in range(_C):
                acc[r, pl.ds(i * 16, 16)] = zero16
            return carry

        lax.fori_loop(0, _CHUNK // 16, zbody, 0)

        lab_lo = labs_v[pl.ds(0, 16)]
        lab_hi = labs_v[pl.ds(16, 16)]
        labs = [lab_lo[i] for i in range(16)] + [lab_hi[i] for i in range(16)]
        cp.wait()

        def bbody(b, carry):
            lab = labs_v[pl.ds(b, 16)][0]
            for j in range(_CHUNK // 16):
                sl = pl.ds(j * 16, 16)
                acc[lab, sl] = acc[lab, sl] + xbuf[b, sl]
            return carry

        lax.fori_loop(0, _B, bbody, 0)
        pltpu.sync_copy(acc, sums_hbm.at[ch, :, pl.ds(qoff, _CHUNK)])

        @pl.when(wid == 0)
        def _():
            cnt = jnp.zeros((16,), jnp.float32)
            for b in range(_B):
                cnt = cnt + jnp.where(iot == labs[b], 1.0, 0.0)
            cntv[...] = cnt
            pltpu.sync_copy(cntv, cnt_hbm)

    return hist(x2d, labi)


def _tc_body(x_ref, s4_ref, cnt_ref, miu_ref, par_ref, out_ref, loss_ref,
             mth_ref, lm_ref, xy0_ref, xy1_ref):
    b = pl.program_id(0)

    @pl.when(b == 0)
    def _stats():
        xw = cnt_ref[...] + _EPS
        w0 = par_ref[33]
        w1 = par_ref[34]
        w2s = par_ref[35]
        mag = (s4_ref[1] + _EPS) / xw
        rot = (s4_ref[0] + _EPS) / xw
        xy0_ref[...] = w2s * ((s4_ref[2] + _EPS) / xw)
        xy1_ref[...] = w2s * ((s4_ref[3] + _EPS) / xw)
        lmag = _lsp(mag + _EPS)
        a_rot = _lsp(rot)
        a_mag = _lsp(mag)
        m_rot = _lsp(miu_ref[:, 0, :])
        m_mag = _lsp(miu_ref[:, 1, :])
        m_mag_e = _lsp(miu_ref[:, 1, :] + _EPS)
        dn = (((1,), (1,)), ((), ()))
        g = (lax.dot_general(a_rot, m_rot, dn,
                             precision=lax.Precision.HIGHEST)
             + lax.dot_general(a_mag, m_mag, dn,
                               precision=lax.Precision.HIGHEST))
        n_a = jnp.sum(a_rot * a_rot + a_mag * a_mag, axis=1,
                      keepdims=True)
        mm = jnp.zeros((_C, _Q), jnp.float32)
        mth0 = jnp.zeros((1, _Q), jnp.float32)
        lossm = jnp.zeros((_D, 128), jnp.float32)
        for d in range(_D):
            al = par_ref[d]
            be = par_ref[8 + d]
            mrd = m_rot[d:d + 1, :]
            mmd = m_mag[d:d + 1, :]
            mth0 = mth0 + be * miu_ref[d, 0:1, :]
            mm = mm + jnp.exp(be * m_mag_e[d:d + 1, :]) * jnp.exp(al * lmag)
            n_m = jnp.sum(mrd * mrd + mmd * mmd)
            nrm = n_a + n_m - 2.0 * g[:, d:d + 1]
            t3 = par_ref[16 + d] / xw
            lossd = par_ref[24 + d] * jnp.mean(nrm + t3)
            rowm = lax.broadcasted_iota(jnp.int32, (_D, 128), 0) == d
            lossm = jnp.where(rowm, lossd, lossm)
        mth_ref[...] = w0 * (par_ref[32] * mag + mth0)
        lm_ref[...] = w1 * jnp.log(mm + _EPS)
        loss_ref[...] = lossm

    w0 = par_ref[33]
    w1 = par_ref[34]
    w2s = par_ref[35]
    mth = mth_ref[...]
    lm = lm_ref[...]
    xy0 = xy0_ref[...]
    xy1 = xy1_ref[...]
    for r in range(8):
        x = x_ref[r:r + 1, :]
        x0 = w0 * x[:, 0:_Q]
        lx1 = w1 * jnp.log(x[:, _Q:2 * _Q])
        x2 = w2s * x[:, 2 * _Q:3 * _Q]
        x3 = w2s * x[:, 3 * _Q:4 * _Q]
        dr = jnp.abs(x0 - mth)
        da = jnp.abs(lx1 - lm)
        dxy = (x2 - xy0) ** 2 + (x3 - xy1) ** 2
        dist = dr + da + dxy
        out_ref[r:r + 1, :] = jnp.min(dist, axis=0, keepdims=True)


def _tc_main(x2d, sums4, cnt, miu2, params):
    return pl.pallas_call(
        _tc_body,
        grid=(_B // 8,),
        in_specs=[
            pl.BlockSpec((8, _P4), lambda g: (g, 0)),
            pl.BlockSpec((4, _C, _Q), lambda g: (0, 0, 0)),
            pl.BlockSpec((_C, 1), lambda g: (0, 0)),
            pl.BlockSpec((_D, 2, _Q), lambda g: (0, 0, 0)),
            pl.BlockSpec(memory_space=pltpu.SMEM),
        ],
        out_specs=[
            pl.BlockSpec((8, _Q), lambda g: (g, 0)),
            pl.BlockSpec((_D, 128), lambda g: (0, 0)),
        ],
        out_shape=[
            jax.ShapeDtypeStruct((_B, _Q), jnp.float32),
            jax.ShapeDtypeStruct((_D, 128), jnp.float32),
        ],
        scratch_shapes=[pltpu.VMEM((_C, _Q), jnp.float32)] * 4,
    )(x2d, sums4, cnt, miu2, params)


def kernel(x_LE, labels, w1, w2, miu, tao, weight):
    x2d = x_LE.reshape(_B, _P4)
    labi = jnp.pad(labels.astype(jnp.int32), (0, 16))
    sums4, cnt = _sc_hist(x2d, labi)

    w1n = w1 ** 2 / jnp.sum(w1 ** 2)
    t2 = tao ** 2
    ft = t2 / (1.0 + t2)
    fs = 1.0 / (1.0 + t2)
    alpha = w1n * ft
    beta = w1n * fs
    t3coef = (2.0 * _Q) * (t2 ** 2 - 1.0)
    term1 = 1.0 / (1.0 + t2) ** 2
    a_sum = jnp.sum(alpha)[None]
    wsq = weight ** 2
    wpar = jnp.stack([wsq[0], wsq[1], jnp.abs(weight[2])])
    params = jnp.concatenate(
        [alpha, beta, t3coef, term1, a_sum, wpar, jnp.zeros((4,), jnp.float32)]
    ).astype(jnp.float32)

    out2, loss_pad = _tc_main(
        x2d, sums4, cnt.reshape(_C, 1), miu.reshape(_D, 2, _Q), params)
    out = out2.reshape(_B, _IN, _H, _W)
    loss = loss_pad[:, 0]
    return out, loss


# speedup vs baseline: 1.2730x; 1.0670x over previous
import functools

import jax
import jax.numpy as jnp
from jax import lax
from jax.experimental import pallas as pl
from jax.experimental.pallas import tpu as pltpu
from jax.experimental.pallas import tpu_sc as plsc

_C = 16
_D = 8
_B = 32
_IN = 64
_H = 16
_W = 16
_OUT = 64
_Q = _OUT * _H * _W
_P4 = 4 * _Q
_EPS = 1e-6
_NW = 32
_CHUNK = _P4 // _NW


def _lsp(x):
    return -jnp.log1p(jnp.exp(-x))


def _sc_hist(x2d, labi):
    mesh = plsc.VectorSubcoreMesh(core_axis_name="c", subcore_axis_name="s")

    @functools.partial(
        pl.kernel,
        out_type=[
            jax.ShapeDtypeStruct((4, _C, _Q), jnp.float32),
            jax.ShapeDtypeStruct((_C,), jnp.float32),
        ],
        mesh=mesh,
        scratch_types=[
            pltpu.VMEM((_B + 16,), jnp.int32),
            pltpu.VMEM((_B, _CHUNK), jnp.float32),
            pltpu.VMEM((_C, _CHUNK), jnp.float32),
            pltpu.VMEM((_C,), jnp.float32),
            pltpu.SemaphoreType.DMA,
        ],
    )
    def hist(x_hbm, lab_hbm, sums_hbm, cnt_hbm, labs_v, xbuf, acc, cntv, sem):
        wid = lax.axis_index("c") * 16 + lax.axis_index("s")
        start = wid * _CHUNK
        ch = start // _Q
        qoff = start % _Q
        cp = pltpu.async_copy(x_hbm.at[:, pl.ds(start, _CHUNK)], xbuf, sem)
        pltpu.sync_copy(lab_hbm, labs_v)
        iot = lax.broadcasted_iota(jnp.int32, (16,), 0)
        zero16 = jnp.zeros((16,), jnp.float32)

        def zbody(i, carry):
            for r in range(_C):
                acc[r, pl.ds(i * 16, 16)] = zero16
            return carry

        lax.fori_loop(0, _CHUNK // 16, zbody, 0)

        lab_lo = labs_v[pl.ds(0, 16)]
        lab_hi = labs_v[pl.ds(16, 16)]
        labs = [lab_lo[i] for i in range(16)] + [lab_hi[i] for i in range(16)]
        cp.wait()

        def bbody(b, carry):
            lab = labs_v[pl.ds(b, 16)][0]
            for g in range(_CHUNK // 128):
                sls = [pl.ds(g * 128 + k * 16, 16) for k in range(8)]
                xs = [xbuf[b, sl] for sl in sls]
                accs = [acc[lab, sl] for sl in sls]
                for k in range(8):
                    acc[lab, sls[k]] = accs[k] + xs[k]
            return carry

        lax.fori_loop(0, _B, bbody, 0)
        pltpu.sync_copy(acc, sums_hbm.at[ch, :, pl.ds(qoff, _CHUNK)])

        @pl.when(wid == 0)
        def _():
            cnt = jnp.zeros((16,), jnp.float32)
            for b in range(_B):
                cnt = cnt + jnp.where(iot == labs[b], 1.0, 0.0)
            cntv[...] = cnt
            pltpu.sync_copy(cntv, cnt_hbm)

    return hist(x2d, labi)


def _tc_body(x_ref, s4_ref, cnt_ref, miu_ref, par_ref, out_ref, loss_ref,
             mth_ref, lm_ref, xy0_ref, xy1_ref):
    b = pl.program_id(0)

    @pl.when(b == 0)
    def _stats():
        xw = cnt_ref[...] + _EPS
        w0 = par_ref[33]
        w1 = par_ref[34]
        w2s = par_ref[35]
        mag = (s4_ref[1] + _EPS) / xw
        rot = (s4_ref[0] + _EPS) / xw
        xy0_ref[...] = w2s * ((s4_ref[2] + _EPS) / xw)
        xy1_ref[...] = w2s * ((s4_ref[3] + _EPS) / xw)
        lmag = _lsp(mag + _EPS)
        a_rot = _lsp(rot)
        a_mag = _lsp(mag)
        miu0 = jnp.concatenate(
            [miu_ref[pl.ds(2 * d * _Q, _Q)][None, :] for d in range(_D)], 0)
        miu1 = jnp.concatenate(
            [miu_ref[pl.ds((2 * d + 1) * _Q, _Q)][None, :] for d in range(_D)],
            0)
        m_rot = _lsp(miu0)
        m_mag = _lsp(miu1)
        m_mag_e = _lsp(miu1 + _EPS)
        dn = (((1,), (1,)), ((), ()))
        g = (lax.dot_general(a_rot, m_rot, dn,
                             precision=lax.Precision.HIGHEST)
             + lax.dot_general(a_mag, m_mag, dn,
                               precision=lax.Precision.HIGHEST))
        n_a = jnp.sum(a_rot * a_rot + a_mag * a_mag, axis=1,
                      keepdims=True)
        mm = jnp.zeros((_C, _Q), jnp.float32)
        mth0 = jnp.zeros((1, _Q), jnp.float32)
        lossm = jnp.zeros((_D, 128), jnp.float32)
        for d in range(_D):
            al = par_ref[d]
            be = par_ref[8 + d]
            mrd = m_rot[d:d + 1, :]
            mmd = m_mag[d:d + 1, :]
            mth0 = mth0 + be * miu0[d:d + 1, :]
            mm = mm + jnp.exp(be * m_mag_e[d:d + 1, :]) * jnp.exp(al * lmag)
            n_m = jnp.sum(mrd * mrd + mmd * mmd)
            nrm = n_a + n_m - 2.0 * g[:, d:d + 1]
            t3 = par_ref[16 + d] / xw
            lossd = par_ref[24 + d] * jnp.mean(nrm + t3)
            rowm = lax.broadcasted_iota(jnp.int32, (_D, 128), 0) == d
            lossm = jnp.where(rowm, lossd, lossm)
        mth_ref[...] = w0 * (par_ref[32] * mag + mth0)
        lm_ref[...] = w1 * jnp.log(mm + _EPS)
        loss_ref[...] = lossm

    w0 = par_ref[33]
    w1 = par_ref[34]
    w2s = par_ref[35]
    mth = mth_ref[...]
    lm = lm_ref[...]
    xy0 = xy0_ref[...]
    xy1 = xy1_ref[...]
    for r in range(8):
        x = x_ref[r:r + 1, :]
        x0 = w0 * x[:, 0:_Q]
        lx1 = w1 * jnp.log(x[:, _Q:2 * _Q])
        x2 = w2s * x[:, 2 * _Q:3 * _Q]
        x3 = w2s * x[:, 3 * _Q:4 * _Q]
        dr = jnp.abs(x0 - mth)
        da = jnp.abs(lx1 - lm)
        dxy = (x2 - xy0) ** 2 + (x3 - xy1) ** 2
        dist = dr + da + dxy
        out_ref[r:r + 1, :] = jnp.min(dist, axis=0, keepdims=True)


def _tc_main(x2d, sums4, cnt, miu2, params):
    return pl.pallas_call(
        _tc_body,
        grid=(_B // 8,),
        in_specs=[
            pl.BlockSpec((8, _P4), lambda g: (g, 0)),
            pl.BlockSpec((4, _C, _Q), lambda g: (0, 0, 0)),
            pl.BlockSpec((_C, 1), lambda g: (0, 0)),
            pl.BlockSpec((2 * _D * _Q,), lambda g: (0,)),
            pl.BlockSpec(memory_space=pltpu.SMEM),
        ],
        out_specs=[
            pl.BlockSpec((8, _Q), lambda g: (g, 0)),
            pl.BlockSpec((_D, 128), lambda g: (0, 0)),
        ],
        out_shape=[
            jax.ShapeDtypeStruct((_B, _Q), jnp.float32),
            jax.ShapeDtypeStruct((_D, 128), jnp.float32),
        ],
        scratch_shapes=[pltpu.VMEM((_C, _Q), jnp.float32)] * 4,
    )(x2d, sums4, cnt, miu2, params)


def kernel(x_LE, labels, w1, w2, miu, tao, weight):
    x2d = x_LE.reshape(_B, _P4)
    labi = jnp.pad(labels.astype(jnp.int32), (0, 16))
    sums4, cnt = _sc_hist(x2d, labi)

    w1n = w1 ** 2 / jnp.sum(w1 ** 2)
    t2 = tao ** 2
    ft = t2 / (1.0 + t2)
    fs = 1.0 / (1.0 + t2)
    alpha = w1n * ft
    beta = w1n * fs
    t3coef = (2.0 * _Q) * (t2 ** 2 - 1.0)
    term1 = 1.0 / (1.0 + t2) ** 2
    a_sum = jnp.sum(alpha)[None]
    wsq = weight ** 2
    wpar = jnp.stack([wsq[0], wsq[1], jnp.abs(weight[2])])
    params = jnp.concatenate(
        [alpha, beta, t3coef, term1, a_sum, wpar, jnp.zeros((4,), jnp.float32)]
    ).astype(jnp.float32)

    out2, loss_pad = _tc_main(
        x2d, sums4, cnt.reshape(_C, 1), miu.reshape(-1), params)
    out = out2.reshape(_B, _IN, _H, _W)
    loss = loss_pad[:, 0]
    return out, loss


# speedup vs baseline: 1.8099x; 1.4218x over previous
import functools

import jax
import jax.numpy as jnp
from jax import lax
from jax.experimental import pallas as pl
from jax.experimental.pallas import tpu as pltpu
from jax.experimental.pallas import tpu_sc as plsc

_C = 16
_D = 8
_B = 32
_IN = 64
_H = 16
_W = 16
_OUT = 64
_Q = _OUT * _H * _W
_P4 = 4 * _Q
_EPS = 1e-6
_NW = 32
_CHUNK = _P4 // _NW


def _lsp(x):
    return -jnp.log1p(jnp.exp(-x))


def _sc_hist(x2d, labi):
    mesh = plsc.VectorSubcoreMesh(core_axis_name="c", subcore_axis_name="s")

    @functools.partial(
        pl.kernel,
        out_type=[
            jax.ShapeDtypeStruct((4, _C, _Q), jnp.float32),
            jax.ShapeDtypeStruct((_C,), jnp.float32),
        ],
        mesh=mesh,
        scratch_types=[
            pltpu.VMEM((_B + 16,), jnp.int32),
            pltpu.VMEM((_B, _CHUNK), jnp.float32),
            pltpu.VMEM((_C, _CHUNK), jnp.float32),
            pltpu.VMEM((_C,), jnp.float32),
            pltpu.SemaphoreType.DMA,
        ],
    )
    def hist(x_hbm, lab_hbm, sums_hbm, cnt_hbm, labs_v, xbuf, acc, cntv, sem):
        wid = lax.axis_index("c") * 16 + lax.axis_index("s")
        start = wid * _CHUNK
        ch = start // _Q
        qoff = start % _Q
        cp = pltpu.async_copy(x_hbm.at[:, pl.ds(start, _CHUNK)], xbuf, sem)
        pltpu.sync_copy(lab_hbm, labs_v)
        iot = lax.broadcasted_iota(jnp.int32, (16,), 0)
        zero16 = jnp.zeros((16,), jnp.float32)

        def zbody(i, carry):
            for r in range(_C):
                acc[r, pl.ds(i * 16, 16)] = zero16
            return carry

        lax.fori_loop(0, _CHUNK // 16, zbody, 0)

        lab_lo = labs_v[pl.ds(0, 16)]
        lab_hi = labs_v[pl.ds(16, 16)]
        labs = [lab_lo[i] for i in range(16)] + [lab_hi[i] for i in range(16)]
        cp.wait()

        def bbody(b, carry):
            lab = labs_v[pl.ds(b, 16)][0]
            for g in range(_CHUNK // 128):
                sls = [pl.ds(g * 128 + k * 16, 16) for k in range(8)]
                xs = [xbuf[b, sl] for sl in sls]
                accs = [acc[lab, sl] for sl in sls]
                for k in range(8):
                    acc[lab, sls[k]] = accs[k] + xs[k]
            return carry

        lax.fori_loop(0, _B, bbody, 0)
        pltpu.sync_copy(acc, sums_hbm.at[ch, :, pl.ds(qoff, _CHUNK)])

        @pl.when(wid == 0)
        def _():
            cnt = jnp.zeros((16,), jnp.float32)
            for b in range(_B):
                cnt = cnt + jnp.where(iot == labs[b], 1.0, 0.0)
            cntv[...] = cnt
            pltpu.sync_copy(cntv, cnt_hbm)

    return hist(x2d, labi)


def _tc_body(x_ref, s4_ref, cnt_ref, miu_ref, par_ref, out_ref, loss_ref,
             mth_ref, lm_ref, xy0_ref, xy1_ref):
    b = pl.program_id(0)

    @pl.when(b == 0)
    def _stats():
        xw = cnt_ref[...] + _EPS
        w0 = par_ref[33]
        w1 = par_ref[34]
        w2s = par_ref[35]
        mag = (s4_ref[1] + _EPS) / xw
        rot = (s4_ref[0] + _EPS) / xw
        xy0_ref[...] = w2s * ((s4_ref[2] + _EPS) / xw)
        xy1_ref[...] = w2s * ((s4_ref[3] + _EPS) / xw)
        lmag = _lsp(mag + _EPS)
        a_rot = _lsp(rot)
        a_mag = _lsp(mag)
        miu0 = jnp.concatenate(
            [miu_ref[pl.ds(2 * d * _Q, _Q)][None, :] for d in range(_D)], 0)
        miu1 = jnp.concatenate(
            [miu_ref[pl.ds((2 * d + 1) * _Q, _Q)][None, :] for d in range(_D)],
            0)
        m_rot = _lsp(miu0)
        m_mag = _lsp(miu1)
        m_mag_e = _lsp(miu1 + _EPS)
        dn = (((1,), (1,)), ((), ()))
        g = (lax.dot_general(a_rot, m_rot, dn,
                             precision=lax.Precision.HIGHEST)
             + lax.dot_general(a_mag, m_mag, dn,
                               precision=lax.Precision.HIGHEST))
        n_a = jnp.sum(a_rot * a_rot + a_mag * a_mag, axis=1,
                      keepdims=True)
        mm = jnp.zeros((_C, _Q), jnp.float32)
        mth0 = jnp.zeros((1, _Q), jnp.float32)
        lossm = jnp.zeros((_D, 128), jnp.float32)
        for d in range(_D):
            al = par_ref[d]
            be = par_ref[8 + d]
            mrd = m_rot[d:d + 1, :]
            mmd = m_mag[d:d + 1, :]
            mth0 = mth0 + be * miu0[d:d + 1, :]
            mm = mm + jnp.exp(be * m_mag_e[d:d + 1, :]) * jnp.exp(al * lmag)
            n_m = jnp.sum(mrd * mrd + mmd * mmd)
            nrm = n_a + n_m - 2.0 * g[:, d:d + 1]
            t3 = par_ref[16 + d] / xw
            lossd = par_ref[24 + d] * jnp.mean(nrm + t3)
            rowm = lax.broadcasted_iota(jnp.int32, (_D, 128), 0) == d
            lossm = jnp.where(rowm, lossd, lossm)
        mth_ref[...] = w0 * (par_ref[32] * mag + mth0)
        lm_ref[...] = w1 * jnp.log(mm + _EPS)
        loss_ref[...] = lossm

    w0 = par_ref[33]
    w1 = par_ref[34]
    w2s = par_ref[35]
    mth = mth_ref[...]
    lm = lm_ref[...]
    xy0 = xy0_ref[...]
    xy1 = xy1_ref[...]
    for r in range(8):
        x = x_ref[r:r + 1, :]
        x0 = w0 * x[:, 0:_Q]
        lx1 = w1 * jnp.log(x[:, _Q:2 * _Q])
        x2 = w2s * x[:, 2 * _Q:3 * _Q]
        x3 = w2s * x[:, 3 * _Q:4 * _Q]
        dr = jnp.abs(x0 - mth)
        da = jnp.abs(lx1 - lm)
        dxy = (x2 - xy0) ** 2 + (x3 - xy1) ** 2
        dist = dr + da + dxy
        out_ref[r:r + 1, :] = jnp.min(dist, axis=0, keepdims=True)


def _tc_main(x2d, sums4, cnt, miu2, params):
    return pl.pallas_call(
        _tc_body,
        grid=(_B // 8,),
        in_specs=[
            pl.BlockSpec((8, _P4), lambda g: (g, 0)),
            pl.BlockSpec((4, _C, _Q), lambda g: (0, 0, 0)),
            pl.BlockSpec((_C, 1), lambda g: (0, 0)),
            pl.BlockSpec((2 * _D * _Q,), lambda g: (0,)),
            pl.BlockSpec(memory_space=pltpu.SMEM),
        ],
        out_specs=[
            pl.BlockSpec((8, _Q), lambda g: (g, 0)),
            pl.BlockSpec((_D, 128), lambda g: (0, 0)),
        ],
        out_shape=[
            jax.ShapeDtypeStruct((_B, _Q), jnp.float32),
            jax.ShapeDtypeStruct((_D, 128), jnp.float32),
        ],
        scratch_shapes=[pltpu.VMEM((_C, _Q), jnp.float32)] * 4,
    )(x2d, sums4, cnt, miu2, params)


def kernel(x_LE, labels, w1, w2, miu, tao, weight):
    x2d = x_LE.transpose(0, 1, 3, 4, 2).reshape(_B, _P4)
    labi = jnp.pad(labels.astype(jnp.int32), (0, 16))
    sums4, cnt = _sc_hist(x2d, labi)

    w1n = w1 ** 2 / jnp.sum(w1 ** 2)
    t2 = tao ** 2
    ft = t2 / (1.0 + t2)
    fs = 1.0 / (1.0 + t2)
    alpha = w1n * ft
    beta = w1n * fs
    t3coef = (2.0 * _Q) * (t2 ** 2 - 1.0)
    term1 = 1.0 / (1.0 + t2) ** 2
    a_sum = jnp.sum(alpha)[None]
    wsq = weight ** 2
    wpar = jnp.stack([wsq[0], wsq[1], jnp.abs(weight[2])])
    params = jnp.concatenate(
        [alpha, beta, t3coef, term1, a_sum, wpar, jnp.zeros((4,), jnp.float32)]
    ).astype(jnp.float32)

    out2, loss_pad = _tc_main(
        x2d, sums4, cnt.reshape(_C, 1),
        miu.transpose(0, 1, 3, 4, 2).reshape(-1), params)
    out = out2.reshape(_B, _H, _W, _IN).transpose(0, 3, 1, 2)
    loss = loss_pad[:, 0]
    return out, loss


# speedup vs baseline: 1.9953x; 1.1025x over previous
import functools

import jax
import jax.numpy as jnp
from jax import lax
from jax.experimental import pallas as pl
from jax.experimental.pallas import tpu as pltpu
from jax.experimental.pallas import tpu_sc as plsc

_C = 16
_D = 8
_B = 32
_IN = 64
_H = 16
_W = 16
_OUT = 64
_Q = _OUT * _H * _W
_P4 = 4 * _Q
_EPS = 1e-6
_NW = 32
_CHUNK = _P4 // _NW


def _lsp(x):
    return -jnp.log1p(jnp.exp(-x))


def _sc_hist(x2d, labi):
    mesh = plsc.VectorSubcoreMesh(core_axis_name="c", subcore_axis_name="s")

    @functools.partial(
        pl.kernel,
        out_type=[
            jax.ShapeDtypeStruct((4, _C, _Q), jnp.float32),
            jax.ShapeDtypeStruct((_C,), jnp.float32),
        ],
        mesh=mesh,
        scratch_types=[
            pltpu.VMEM((_B + 16,), jnp.int32),
            pltpu.VMEM((_B, _CHUNK), jnp.float32),
            pltpu.VMEM((_C, _CHUNK), jnp.float32),
            pltpu.VMEM((_C,), jnp.float32),
            pltpu.SemaphoreType.DMA,
        ],
    )
    def hist(x_hbm, lab_hbm, sums_hbm, cnt_hbm, labs_v, xbuf, acc, cntv, sem):
        wid = lax.axis_index("c") * 16 + lax.axis_index("s")
        start = wid * _CHUNK
        ch = start // _Q
        qoff = start % _Q
        cp = pltpu.async_copy(x_hbm.at[:, pl.ds(start, _CHUNK)], xbuf, sem)
        pltpu.sync_copy(lab_hbm, labs_v)
        iot = lax.broadcasted_iota(jnp.int32, (16,), 0)
        zero16 = jnp.zeros((16,), jnp.float32)

        def zbody(i, carry):
            for r in range(_C):
                acc[r, pl.ds(i * 16, 16)] = zero16
            return carry

        lax.fori_loop(0, _CHUNK // 16, zbody, 0)

        lab_lo = labs_v[pl.ds(0, 16)]
        lab_hi = labs_v[pl.ds(16, 16)]
        labs = [lab_lo[i] for i in range(16)] + [lab_hi[i] for i in range(16)]
        cp.wait()

        def bbody(b, carry):
            lab = labs_v[pl.ds(b, 16)][0]
            for g in range(_CHUNK // 128):
                sls = [pl.ds(g * 128 + k * 16, 16) for k in range(8)]
                xs = [xbuf[b, sl] for sl in sls]
                accs = [acc[lab, sl] for sl in sls]
                for k in range(8):
                    acc[lab, sls[k]] = accs[k] + xs[k]
            return carry

        lax.fori_loop(0, _B, bbody, 0)
        pltpu.sync_copy(acc, sums_hbm.at[ch, :, pl.ds(qoff, _CHUNK)])

        @pl.when(wid == 0)
        def _():
            cnt = jnp.zeros((16,), jnp.float32)
            for b in range(_B):
                cnt = cnt + jnp.where(iot == labs[b], 1.0, 0.0)
            cntv[...] = cnt
            pltpu.sync_copy(cntv, cnt_hbm)

    return hist(x2d, labi)


def _tc_body(x_ref, s4_ref, cnt_ref, miu_ref, par_ref, parv_ref, out_ref,
             loss_ref, mth_ref, lm_ref, xy0_ref, xy1_ref):
    b = pl.program_id(0)

    @pl.when(b == 0)
    def _stats():
        xw = cnt_ref[...] + _EPS
        w0 = par_ref[33]
        w1 = par_ref[34]
        w2s = par_ref[35]
        mag = (s4_ref[1] + _EPS) / xw
        rot = (s4_ref[0] + _EPS) / xw
        lmag = _lsp(mag + _EPS)
        a_rot = _lsp(rot)
        a_mag = _lsp(mag)
        miu0 = jnp.concatenate(
            [miu_ref[pl.ds(2 * d * _Q, _Q)][None, :] for d in range(_D)], 0)
        miu1 = jnp.concatenate(
            [miu_ref[pl.ds((2 * d + 1) * _Q, _Q)][None, :] for d in range(_D)],
            0)
        m_rot = _lsp(miu0)
        m_mag = _lsp(miu1)
        m_mag_e = _lsp(miu1 + _EPS)
        dn = (((1,), (1,)), ((), ()))
        g = (lax.dot_general(a_rot, m_rot, dn,
                             precision=lax.Precision.HIGHEST)
             + lax.dot_general(a_mag, m_mag, dn,
                               precision=lax.Precision.HIGHEST))
        n_a = jnp.sum(a_rot * a_rot + a_mag * a_mag, axis=1,
                      keepdims=True)
        mm = jnp.zeros((_C, _Q), jnp.float32)
        lossm = jnp.zeros((_D, 128), jnp.float32)
        beta_row = parv_ref[:, 8:16]
        mth0 = lax.dot_general(beta_row, miu0, (((1,), (0,)), ((), ())),
                               precision=lax.Precision.HIGHEST)
        for d in range(_D):
            al = par_ref[d]
            be = par_ref[8 + d]
            mrd = m_rot[d:d + 1, :]
            mmd = m_mag[d:d + 1, :]
            mm = mm + jnp.exp(be * m_mag_e[d:d + 1, :]) * jnp.exp(al * lmag)
            n_m = jnp.sum(mrd * mrd + mmd * mmd)
            nrm = n_a + n_m - 2.0 * g[:, d:d + 1]
            t3 = par_ref[16 + d] / xw
            lossd = par_ref[24 + d] * jnp.mean(nrm + t3)
            rowm = lax.broadcasted_iota(jnp.int32, (_D, 128), 0) == d
            lossm = jnp.where(rowm, lossd, lossm)
        mth = w0 * (par_ref[32] * mag + mth0)
        lm = w1 * jnp.log(mm + _EPS)
        xy0 = w2s * ((s4_ref[2] + _EPS) / xw)
        xy1 = w2s * ((s4_ref[3] + _EPS) / xw)
        for c in range(_C):
            mth_ref[c] = jnp.broadcast_to(mth[c:c + 1, :], (8, _Q))
            lm_ref[c] = jnp.broadcast_to(lm[c:c + 1, :], (8, _Q))
            xy0_ref[c] = jnp.broadcast_to(xy0[c:c + 1, :], (8, _Q))
            xy1_ref[c] = jnp.broadcast_to(xy1[c:c + 1, :], (8, _Q))
        loss_ref[...] = lossm

    w0 = par_ref[33]
    w1 = par_ref[34]
    w2s = par_ref[35]
    x0a = w0 * x_ref[:, 0:_Q]
    lx1a = w1 * jnp.log(x_ref[:, _Q:2 * _Q])
    x2a = w2s * x_ref[:, 2 * _Q:3 * _Q]
    x3a = w2s * x_ref[:, 3 * _Q:4 * _Q]
    m = None
    for c in range(_C):
        dc = (jnp.abs(x0a - mth_ref[c]) + jnp.abs(lx1a - lm_ref[c])
              + (x2a - xy0_ref[c]) ** 2 + (x3a - xy1_ref[c]) ** 2)
        m = dc if m is None else jnp.minimum(m, dc)
    out_ref[...] = m


def _tc_main(x2d, sums4, cnt, miu2, params):
    return pl.pallas_call(
        _tc_body,
        grid=(_B // 8,),
        in_specs=[
            pl.BlockSpec((8, _P4), lambda g: (g, 0)),
            pl.BlockSpec((4, _C, _Q), lambda g: (0, 0, 0)),
            pl.BlockSpec((_C, 1), lambda g: (0, 0)),
            pl.BlockSpec((2 * _D * _Q,), lambda g: (0,)),
            pl.BlockSpec(memory_space=pltpu.SMEM),
            pl.BlockSpec((1, 40), lambda g: (0, 0)),
        ],
        out_specs=[
            pl.BlockSpec((8, _Q), lambda g: (g, 0)),
            pl.BlockSpec((_D, 128), lambda g: (0, 0)),
        ],
        out_shape=[
            jax.ShapeDtypeStruct((_B, _Q), jnp.float32),
            jax.ShapeDtypeStruct((_D, 128), jnp.float32),
        ],
        scratch_shapes=[pltpu.VMEM((_C, 8, _Q), jnp.float32)] * 4,
    )(x2d, sums4, cnt, miu2, params, params.reshape(1, 40))


def kernel(x_LE, labels, w1, w2, miu, tao, weight):
    x2d = x_LE.transpose(0, 1, 3, 4, 2).reshape(_B, _P4)
    labi = jnp.pad(labels.astype(jnp.int32), (0, 16))
    sums4, cnt = _sc_hist(x2d, labi)

    w1n = w1 ** 2 / jnp.sum(w1 ** 2)
    t2 = tao ** 2
    ft = t2 / (1.0 + t2)
    fs = 1.0 / (1.0 + t2)
    alpha = w1n * ft
    beta = w1n * fs
    t3coef = (2.0 * _Q) * (t2 ** 2 - 1.0)
    term1 = 1.0 / (1.0 + t2) ** 2
    a_sum = jnp.sum(alpha)[None]
    wsq = weight ** 2
    wpar = jnp.stack([wsq[0], wsq[1], jnp.abs(weight[2])])
    params = jnp.concatenate(
        [alpha, beta, t3coef, term1, a_sum, wpar, jnp.zeros((4,), jnp.float32)]
    ).astype(jnp.float32)

    out2, loss_pad = _tc_main(
        x2d, sums4, cnt.reshape(_C, 1),
        miu.transpose(0, 1, 3, 4, 2).reshape(-1), params)
    out = out2.reshape(_B, _H, _W, _IN).transpose(0, 3, 1, 2)
    loss = loss_pad[:, 0]
    return out, loss


# speedup vs baseline: 2.0363x; 1.0206x over previous
import functools

import jax
import jax.numpy as jnp
from jax import lax
from jax.experimental import pallas as pl
from jax.experimental.pallas import tpu as pltpu
from jax.experimental.pallas import tpu_sc as plsc

_C = 16
_D = 8
_B = 32
_IN = 64
_H = 16
_W = 16
_OUT = 64
_Q = _OUT * _H * _W
_P4 = 4 * _Q
_EPS = 1e-6
_NW = 32
_CHUNK = _P4 // _NW


def _lsp(x):
    return -jnp.log1p(jnp.exp(-x))


def _sc_hist(x2d, labi):
    mesh = plsc.VectorSubcoreMesh(core_axis_name="c", subcore_axis_name="s")

    half = _CHUNK // 2

    @functools.partial(
        pl.kernel,
        out_type=jax.ShapeDtypeStruct((4, _C, _Q), jnp.float32),
        mesh=mesh,
        scratch_types=[
            pltpu.VMEM((_B + 16,), jnp.int32),
            pltpu.VMEM((_B, _CHUNK), jnp.float32),
            pltpu.VMEM((_C, _CHUNK), jnp.float32),
            pltpu.SemaphoreType.DMA,
            pltpu.SemaphoreType.DMA,
        ],
    )
    def hist(x_hbm, lab_hbm, sums_hbm, labs_v, xbuf, acc, sem0, sem1):
        wid = lax.axis_index("c") * 16 + lax.axis_index("s")
        start = wid * _CHUNK
        ch = start // _Q
        qoff = start % _Q
        cp0 = pltpu.async_copy(x_hbm.at[:, pl.ds(start, half)],
                               xbuf.at[:, pl.ds(0, half)], sem0)
        cp1 = pltpu.async_copy(x_hbm.at[:, pl.ds(start + half, half)],
                               xbuf.at[:, pl.ds(half, half)], sem1)
        pltpu.sync_copy(lab_hbm, labs_v)
        zero16 = jnp.zeros((16,), jnp.float32)

        def zbody(i, carry):
            for r in range(_C):
                acc[r, pl.ds(i * 16, 16)] = zero16
            return carry

        lax.fori_loop(0, _CHUNK // 16, zbody, 0)

        def make_body(col0, ngrp):
            def bbody(b, carry):
                lab = labs_v[pl.ds(b, 16)][0]
                for g in range(ngrp):
                    sls = [pl.ds(col0 + g * 128 + k * 16, 16)
                           for k in range(8)]
                    xs = [xbuf[b, sl] for sl in sls]
                    accs = [acc[lab, sl] for sl in sls]
                    for k in range(8):
                        acc[lab, sls[k]] = accs[k] + xs[k]
                return carry
            return bbody

        cp0.wait()
        lax.fori_loop(0, _B, make_body(0, half // 128), 0)
        cp1.wait()
        lax.fori_loop(0, _B, make_body(half, half // 128), 0)
        pltpu.sync_copy(acc, sums_hbm.at[ch, :, pl.ds(qoff, _CHUNK)])

    return hist(x2d, labi)


def _tc_body(x_ref, s4_ref, lab_ref, miu_ref, par_ref, parv_ref, out_ref,
             loss_ref, mth_ref, lm_ref, xy0_ref, xy1_ref):
    b = pl.program_id(0)

    @pl.when(b == 0)
    def _stats():
        oh = jnp.where(
            lab_ref[...] == lax.broadcasted_iota(jnp.int32, (_C, _B), 0),
            1.0, 0.0)
        xw = jnp.sum(oh, axis=1, keepdims=True) + _EPS
        w0 = par_ref[33]
        w1 = par_ref[34]
        w2s = par_ref[35]
        mag = (s4_ref[1] + _EPS) / xw
        rot = (s4_ref[0] + _EPS) / xw
        lmag = _lsp(mag + _EPS)
        a_rot = _lsp(rot)
        a_mag = lmag
        miu0 = jnp.concatenate(
            [miu_ref[pl.ds(2 * d * _Q, _Q)][None, :] for d in range(_D)], 0)
        miu1 = jnp.concatenate(
            [miu_ref[pl.ds((2 * d + 1) * _Q, _Q)][None, :] for d in range(_D)],
            0)
        m_rot = _lsp(miu0)
        m_mag = _lsp(miu1)
        m_mag_e = m_mag
        dn = (((1,), (1,)), ((), ()))
        g = (lax.dot_general(a_rot, m_rot, dn,
                             precision=lax.Precision.HIGHEST)
             + lax.dot_general(a_mag, m_mag, dn,
                               precision=lax.Precision.HIGHEST))
        n_a = jnp.sum(a_rot * a_rot + a_mag * a_mag, axis=1,
                      keepdims=True)
        mm = jnp.zeros((_C, _Q), jnp.float32)
        lossm = jnp.zeros((_D, 128), jnp.float32)
        beta_row = parv_ref[:, 8:16]
        mth0 = lax.dot_general(beta_row, miu0, (((1,), (0,)), ((), ())),
                               precision=lax.Precision.HIGHEST)
        for d in range(_D):
            al = par_ref[d]
            be = par_ref[8 + d]
            mrd = m_rot[d:d + 1, :]
            mmd = m_mag[d:d + 1, :]
            mm = mm + jnp.exp(be * m_mag_e[d:d + 1, :]) * jnp.exp(al * lmag)
            n_m = jnp.sum(mrd * mrd + mmd * mmd)
            nrm = n_a + n_m - 2.0 * g[:, d:d + 1]
            t3 = par_ref[16 + d] / xw
            lossd = par_ref[24 + d] * jnp.mean(nrm + t3)
            rowm = lax.broadcasted_iota(jnp.int32, (_D, 128), 0) == d
            lossm = jnp.where(rowm, lossd, lossm)
        mth = w0 * (par_ref[32] * mag + mth0)
        lm = w1 * jnp.log(mm + _EPS)
        xy0 = w2s * ((s4_ref[2] + _EPS) / xw)
        xy1 = w2s * ((s4_ref[3] + _EPS) / xw)
        for c in range(_C):
            mth_ref[c] = jnp.broadcast_to(mth[c:c + 1, :], (8, _Q))
            lm_ref[c] = jnp.broadcast_to(lm[c:c + 1, :], (8, _Q))
            xy0_ref[c] = jnp.broadcast_to(xy0[c:c + 1, :], (8, _Q))
            xy1_ref[c] = jnp.broadcast_to(xy1[c:c + 1, :], (8, _Q))
        loss_ref[...] = lossm

    w0 = par_ref[33]
    w1 = par_ref[34]
    w2s = par_ref[35]
    x0a = w0 * x_ref[:, 0:_Q]
    lx1a = w1 * jnp.log(x_ref[:, _Q:2 * _Q])
    x2a = w2s * x_ref[:, 2 * _Q:3 * _Q]
    x3a = w2s * x_ref[:, 3 * _Q:4 * _Q]
    m = None
    for c in range(_C):
        dc = (jnp.abs(x0a - mth_ref[c]) + jnp.abs(lx1a - lm_ref[c])
              + (x2a - xy0_ref[c]) ** 2 + (x3a - xy1_ref[c]) ** 2)
        m = dc if m is None else jnp.minimum(m, dc)
    out_ref[...] = m


def _tc_main(x2d, sums4, lab2, miu2, params):
    return pl.pallas_call(
        _tc_body,
        grid=(_B // 8,),
        in_specs=[
            pl.BlockSpec((8, _P4), lambda g: (g, 0)),
            pl.BlockSpec((4, _C, _Q), lambda g: (0, 0, 0)),
            pl.BlockSpec((1, _B), lambda g: (0, 0)),
            pl.BlockSpec((2 * _D * _Q,), lambda g: (0,)),
            pl.BlockSpec(memory_space=pltpu.SMEM),
            pl.BlockSpec((1, 40), lambda g: (0, 0)),
        ],
        out_specs=[
            pl.BlockSpec((8, _Q), lambda g: (g, 0)),
            pl.BlockSpec((_D, 128), lambda g: (0, 0)),
        ],
        out_shape=[
            jax.ShapeDtypeStruct((_B, _Q), jnp.float32),
            jax.ShapeDtypeStruct((_D, 128), jnp.float32),
        ],
        scratch_shapes=[pltpu.VMEM((_C, 8, _Q), jnp.float32)] * 4,
    )(x2d, sums4, lab2, miu2, params, params.reshape(1, 40))


def kernel(x_LE, labels, w1, w2, miu, tao, weight):
    x2d = x_LE.transpose(0, 1, 3, 4, 2).reshape(_B, _P4)
    labi32 = labels.astype(jnp.int32)
    sums4 = _sc_hist(x2d, jnp.pad(labi32, (0, 16)))

    w1n = w1 ** 2 / jnp.sum(w1 ** 2)
    t2 = tao ** 2
    ft = t2 / (1.0 + t2)
    fs = 1.0 / (1.0 + t2)
    alpha = w1n * ft
    beta = w1n * fs
    t3coef = (2.0 * _Q) * (t2 ** 2 - 1.0)
    term1 = 1.0 / (1.0 + t2) ** 2
    a_sum = jnp.sum(alpha)[None]
    wsq = weight ** 2
    wpar = jnp.stack([wsq[0], wsq[1], jnp.abs(weight[2])])
    params = jnp.concatenate(
        [alpha, beta, t3coef, term1, a_sum, wpar, jnp.zeros((4,), jnp.float32)]
    ).astype(jnp.float32)

    out2, loss_pad = _tc_main(
        x2d, sums4, labi32.reshape(1, _B),
        miu.transpose(0, 1, 3, 4, 2).reshape(-1), params)
    out = out2.reshape(_B, _H, _W, _IN).transpose(0, 3, 1, 2)
    loss = loss_pad[:, 0]
    return out, loss


# speedup vs baseline: 2.0376x; 1.0006x over previous
import functools

import jax
import jax.numpy as jnp
from jax import lax
from jax.experimental import pallas as pl
from jax.experimental.pallas import tpu as pltpu
from jax.experimental.pallas import tpu_sc as plsc

_C = 16
_D = 8
_B = 32
_IN = 64
_H = 16
_W = 16
_OUT = 64
_Q = _OUT * _H * _W
_P4 = 4 * _Q
_EPS = 1e-6
_NW = 32
_CHUNK = _P4 // _NW


def _lsp(x):
    return -jnp.log1p(jnp.exp(-x))


def _sc_hist(x2d, labi):
    mesh = plsc.VectorSubcoreMesh(core_axis_name="c", subcore_axis_name="s")


    half = _CHUNK // 2

    @functools.partial(
        pl.kernel,
        out_type=jax.ShapeDtypeStruct((4, _C, _Q), jnp.float32),
        mesh=mesh,
        scratch_types=[
            pltpu.VMEM((_B + 16,), jnp.int32),
            pltpu.VMEM((_B, _CHUNK), jnp.float32),
            pltpu.VMEM((_C, _CHUNK), jnp.float32),
            pltpu.SemaphoreType.DMA,
            pltpu.SemaphoreType.DMA,
        ],
    )
    def hist(x_hbm, lab_hbm, sums_hbm, labs_v, xbuf, acc, sem0, sem1):
        wid = lax.axis_index("c") * 16 + lax.axis_index("s")
        start = wid * _CHUNK
        ch = start // _Q
        qoff = start % _Q
        cp0 = pltpu.async_copy(x_hbm.at[:, pl.ds(start, half)],
                               xbuf.at[:, pl.ds(0, half)], sem0)
        cp1 = pltpu.async_copy(x_hbm.at[:, pl.ds(start + half, half)],
                               xbuf.at[:, pl.ds(half, half)], sem1)
        pltpu.sync_copy(lab_hbm, labs_v)
        zero16 = jnp.zeros((16,), jnp.float32)

        def zbody(i, carry):
            for r in range(_C):
                for k in range(4):
                    acc[r, pl.ds(i * 64 + k * 16, 16)] = zero16
            return carry

        lax.fori_loop(0, _CHUNK // 64, zbody, 0)

        def make_body(col0, ngrp):
            def bbody(b, carry):
                lab = labs_v[pl.ds(b, 16)][0]
                for g in range(ngrp):
                    sls = [pl.ds(col0 + g * 128 + k * 16, 16)
                           for k in range(8)]
                    xs = [xbuf[b, sl] for sl in sls]
                    accs = [acc[lab, sl] for sl in sls]
                    for k in range(8):
                        acc[lab, sls[k]] = accs[k] + xs[k]
                return carry
            return bbody

        cp0.wait()
        lax.fori_loop(0, _B, make_body(0, half // 128), 0)
        cp1.wait()
        lax.fori_loop(0, _B, make_body(half, half // 128), 0)
        pltpu.sync_copy(acc, sums_hbm.at[ch, :, pl.ds(qoff, _CHUNK)])

    return hist(x2d, labi)


def _tc_body(x_ref, s4_ref, lab_ref, miu_ref, par_ref, parv_ref, out_ref,
             loss_ref, mth_ref, lm_ref, xy0_ref, xy1_ref):
    b = pl.program_id(0)

    @pl.when(b == 0)
    def _stats():
        oh = jnp.where(
            lab_ref[...] == lax.broadcasted_iota(jnp.int32, (_C, _B), 0),
            1.0, 0.0)
        xw = jnp.sum(oh, axis=1, keepdims=True) + _EPS
        w0 = par_ref[33]
        w1 = par_ref[34]
        w2s = par_ref[35]
        mag = (s4_ref[1] + _EPS) / xw
        rot = (s4_ref[0] + _EPS) / xw
        lmag = _lsp(mag + _EPS)
        a_rot = _lsp(rot)
        a_mag = lmag
        miu0 = jnp.concatenate(
            [miu_ref[pl.ds(2 * d * _Q, _Q)][None, :] for d in range(_D)], 0)
        miu1 = jnp.concatenate(
            [miu_ref[pl.ds((2 * d + 1) * _Q, _Q)][None, :] for d in range(_D)],
            0)
        m_rot = _lsp(miu0)
        m_mag = _lsp(miu1)
        m_mag_e = m_mag
        dn = (((1,), (1,)), ((), ()))
        g = (lax.dot_general(a_rot, m_rot, dn,
                             precision=lax.Precision.HIGHEST)
             + lax.dot_general(a_mag, m_mag, dn,
                               precision=lax.Precision.HIGHEST))
        n_a = jnp.sum(a_rot * a_rot + a_mag * a_mag, axis=1,
                      keepdims=True)
        mm = jnp.zeros((_C, _Q), jnp.float32)
        lossm = jnp.zeros((_D, 128), jnp.float32)
        beta_row = parv_ref[:, 8:16]
        mth0 = lax.dot_general(beta_row, miu0, (((1,), (0,)), ((), ())),
                               precision=lax.Precision.HIGHEST)
        for d in range(_D):
            al = par_ref[d]
            be = par_ref[8 + d]
            mrd = m_rot[d:d + 1, :]
            mmd = m_mag[d:d + 1, :]
            mm = mm + jnp.exp(be * m_mag_e[d:d + 1, :]) * jnp.exp(al * lmag)
            n_m = jnp.sum(mrd * mrd + mmd * mmd)
            nrm = n_a + n_m - 2.0 * g[:, d:d + 1]
            t3 = par_ref[16 + d] / xw
            lossd = par_ref[24 + d] * jnp.mean(nrm + t3)
            rowm = lax.broadcasted_iota(jnp.int32, (_D, 128), 0) == d
            lossm = jnp.where(rowm, lossd, lossm)
        mth = w0 * (par_ref[32] * mag + mth0)
        lm = w1 * jnp.log(mm + _EPS)
        xy0 = w2s * ((s4_ref[2] + _EPS) / xw)
        xy1 = w2s * ((s4_ref[3] + _EPS) / xw)
        for c in range(_C):
            mth_ref[c] = jnp.broadcast_to(mth[c:c + 1, :], (8, _Q))
            lm_ref[c] = jnp.broadcast_to(lm[c:c + 1, :], (8, _Q))
            xy0_ref[c] = jnp.broadcast_to(xy0[c:c + 1, :], (8, _Q))
            xy1_ref[c] = jnp.broadcast_to(xy1[c:c + 1, :], (8, _Q))
        loss_ref[...] = lossm

    w0 = par_ref[33]
    w1 = par_ref[34]
    w2s = par_ref[35]
    x0a = w0 * x_ref[:, 0:_Q]
    lx1a = w1 * jnp.log(x_ref[:, _Q:2 * _Q])
    x2a = w2s * x_ref[:, 2 * _Q:3 * _Q]
    x3a = w2s * x_ref[:, 3 * _Q:4 * _Q]
    m = None
    for c in range(_C):
        dc = (jnp.abs(x0a - mth_ref[c]) + jnp.abs(lx1a - lm_ref[c])
              + (x2a - xy0_ref[c]) ** 2 + (x3a - xy1_ref[c]) ** 2)
        m = dc if m is None else jnp.minimum(m, dc)
    out_ref[...] = m


def _tc_main(x2d, sums4, lab2, miu2, params):
    return pl.pallas_call(
        _tc_body,
        grid=(_B // 8,),
        in_specs=[
            pl.BlockSpec((8, _P4), lambda g: (g, 0)),
            pl.BlockSpec((4, _C, _Q), lambda g: (0, 0, 0)),
            pl.BlockSpec((1, _B), lambda g: (0, 0)),
            pl.BlockSpec((2 * _D * _Q,), lambda g: (0,)),
            pl.BlockSpec(memory_space=pltpu.SMEM),
            pl.BlockSpec((1, 40), lambda g: (0, 0)),
        ],
        out_specs=[
            pl.BlockSpec((8, _Q), lambda g: (g, 0)),
            pl.BlockSpec((_D, 128), lambda g: (0, 0)),
        ],
        out_shape=[
            jax.ShapeDtypeStruct((_B, _Q), jnp.float32),
            jax.ShapeDtypeStruct((_D, 128), jnp.float32),
        ],
        scratch_shapes=[pltpu.VMEM((_C, 8, _Q), jnp.float32)] * 4,
    )(x2d, sums4, lab2, miu2, params, params.reshape(1, 40))


def kernel(x_LE, labels, w1, w2, miu, tao, weight):
    x2d = x_LE.transpose(0, 1, 3, 4, 2).reshape(_B, _P4)
    labi32 = labels.astype(jnp.int32)
    sums4 = _sc_hist(x2d, jnp.pad(labi32, (0, 16)))

    w1n = w1 ** 2 / jnp.sum(w1 ** 2)
    t2 = tao ** 2
    ft = t2 / (1.0 + t2)
    fs = 1.0 / (1.0 + t2)
    alpha = w1n * ft
    beta = w1n * fs
    t3coef = (2.0 * _Q) * (t2 ** 2 - 1.0)
    term1 = 1.0 / (1.0 + t2) ** 2
    a_sum = jnp.sum(alpha)[None]
    wsq = weight ** 2
    wpar = jnp.stack([wsq[0], wsq[1], jnp.abs(weight[2])])
    params = jnp.concatenate(
        [alpha, beta, t3coef, term1, a_sum, wpar, jnp.zeros((4,), jnp.float32)]
    ).astype(jnp.float32)

    out2, loss_pad = _tc_main(
        x2d, sums4, labi32.reshape(1, _B),
        miu.transpose(0, 1, 3, 4, 2).reshape(-1), params)
    out = out2.reshape(_B, _H, _W, _IN).transpose(0, 3, 1, 2)
    loss = loss_pad[:, 0]
    return out, loss
